# baseline (speedup 1.0000x reference)
import functools

import jax
import jax.numpy as jnp
from jax import lax
from jax.experimental import pallas as pl
from jax.experimental.pallas import tpu as pltpu
from jax.experimental.pallas import tpu_sc as plsc

V = 100000
H = 64
B = 4096


_NC, _NS = 2, 16
_NW = _NC * _NS
_B_PER_W = B // _NW


def _gather_body(table_hbm, idx_hbm, out_hbm, idx_v, rows_v, sem):
    wid = lax.axis_index("s") * _NC + lax.axis_index("c")
    base = wid * _B_PER_W
    pltpu.sync_copy(idx_hbm.at[pl.ds(base, _B_PER_W)], idx_v)
    pltpu.async_copy(table_hbm.at[idx_v], rows_v, sem).wait()
    pltpu.sync_copy(rows_v, out_hbm.at[pl.ds(base, _B_PER_W)])


@functools.cache
def _sc_gather():
    return pl.kernel(
        _gather_body,
        out_type=jax.ShapeDtypeStruct((B, H), jnp.float32),
        mesh=plsc.VectorSubcoreMesh(
            core_axis_name="c", subcore_axis_name="s",
            num_cores=_NC, num_subcores=_NS,
        ),
        scratch_types=[
            pltpu.VMEM((_B_PER_W,), jnp.int32),
            pltpu.VMEM((_B_PER_W, H), jnp.float32),
            pltpu.SemaphoreType.DMA,
        ],
        compiler_params=pltpu.CompilerParams(use_tc_tiling_on_sc=False),
    )


_VBLK = 512
_NBUF = 3
_NV = 195
_VMAIN = _NV * _VBLK
_TAIL = V - _VMAIN


def _head_body(h_ref, w_ref, b_ref, out_hbm, obuf, sem):
    j = pl.program_id(0)
    slot = lax.rem(j, _NBUF)

    @pl.when(j >= _NBUF)
    def _wait_slot():
        pltpu.make_async_copy(
            obuf.at[slot],
            out_hbm.at[:, pl.ds((j - _NBUF) * _VBLK, _VBLK)],
            sem.at[slot],
        ).wait()

    acc = lax.dot_general(
        h_ref[...],
        w_ref[...],
        (((1,), (1,)), ((), ())),
        preferred_element_type=jnp.float32,
    )
    obuf[slot] = acc + b_ref[...]

    pltpu.make_async_copy(
        obuf.at[slot],
        out_hbm.at[:, pl.ds(j * _VBLK, _VBLK)],
        sem.at[slot],
    ).start()

    @pl.when(j == _NV - 1)
    def _drain():
        for i in range(_NBUF):
            pltpu.make_async_copy(
                obuf.at[i],
                out_hbm.at[:, pl.ds(i * _VBLK, _VBLK)],
                sem.at[i],
            ).wait()


def _head(h, head_weight, bias2d):
    return pl.pallas_call(
        _head_body,
        grid=(_NV,),
        in_specs=[
            pl.BlockSpec((B, H), lambda j: (0, 0)),
            pl.BlockSpec((_VBLK, H), lambda j: (j, 0)),
            pl.BlockSpec((1, _VBLK), lambda j: (0, j)),
        ],
        out_specs=pl.BlockSpec(memory_space=pl.ANY),
        out_shape=jax.ShapeDtypeStruct((B, V), jnp.float32),
        scratch_shapes=[
            pltpu.VMEM((_NBUF, B, _VBLK), jnp.float32),
            pltpu.SemaphoreType.DMA((_NBUF,)),
        ],
        compiler_params=pltpu.CompilerParams(
            dimension_semantics=("arbitrary",),
        ),
    )(h, head_weight, bias2d)


def _tail_body(out_in, h_ref, w_ref, b_ref, out_ref):
    del out_in
    acc = lax.dot_general(
        h_ref[...],
        w_ref[...],
        (((1,), (1,)), ((), ())),
        preferred_element_type=jnp.float32,
    )
    out_ref[...] = acc + b_ref[...]


def _tail(logits, h, head_weight, bias2d):
    return pl.pallas_call(
        _tail_body,
        grid=(1,),
        in_specs=[
            pl.BlockSpec(memory_space=pl.ANY),
            pl.BlockSpec((B, H), lambda j: (0, 0)),
            pl.BlockSpec((_VBLK, H), lambda j: (_NV, 0)),
            pl.BlockSpec((1, _VBLK), lambda j: (0, _NV)),
        ],
        out_specs=pl.BlockSpec((B, _VBLK), lambda j: (0, _NV)),
        out_shape=jax.ShapeDtypeStruct((B, V), jnp.float32),
        input_output_aliases={0: 0},
    )(logits, h, head_weight, bias2d)


def kernel(input_ids, embed_weight, head_weight, head_bias):
    last_ids = input_ids[:, -1]
    h = _sc_gather()(embed_weight, last_ids)
    bias2d = head_bias.reshape(1, V)
    logits = _head(h, head_weight, bias2d)
    logits = _tail(logits, h, head_weight, bias2d)
    return logits


# speedup vs baseline: 1.0006x; 1.0006x over previous
import functools

import jax
import jax.numpy as jnp
from jax import lax
from jax.experimental import pallas as pl
from jax.experimental.pallas import tpu as pltpu
from jax.experimental.pallas import tpu_sc as plsc

V = 100000
H = 64
B = 4096


_NC, _NS = 2, 16
_NW = _NC * _NS
_B_PER_W = B // _NW


def _gather_body(table_hbm, idx_hbm, out_hbm, idx_v, rows_v, sem):
    wid = lax.axis_index("s") * _NC + lax.axis_index("c")
    base = wid * _B_PER_W
    pltpu.sync_copy(idx_hbm.at[pl.ds(base, _B_PER_W)], idx_v)
    pltpu.async_copy(table_hbm.at[idx_v], rows_v, sem).wait()
    pltpu.sync_copy(rows_v, out_hbm.at[pl.ds(base, _B_PER_W)])


@functools.cache
def _sc_gather():
    return pl.kernel(
        _gather_body,
        out_type=jax.ShapeDtypeStruct((B, H), jnp.float32),
        mesh=plsc.VectorSubcoreMesh(
            core_axis_name="c", subcore_axis_name="s",
            num_cores=_NC, num_subcores=_NS,
        ),
        scratch_types=[
            pltpu.VMEM((_B_PER_W,), jnp.int32),
            pltpu.VMEM((_B_PER_W, H), jnp.float32),
            pltpu.SemaphoreType.DMA,
        ],
        compiler_params=pltpu.CompilerParams(use_tc_tiling_on_sc=False),
    )


_VBLK = 512
_NBUF = 3
_NSPLIT = 4
_RSPL = B // _NSPLIT
_NV = 195
_VMAIN = _NV * _VBLK
_TAIL = V - _VMAIN


def _head_body(h_ref, w_ref, b_ref, out_hbm, obuf, sem):
    j = pl.program_id(0)
    slot = lax.rem(j, _NBUF)

    @pl.when(j >= _NBUF)
    def _wait_slot():
        pltpu.make_async_copy(
            obuf.at[slot],
            out_hbm.at[:, pl.ds((j - _NBUF) * _VBLK, _VBLK)],
            sem.at[slot],
        ).wait()

    acc = lax.dot_general(
        h_ref[...],
        w_ref[...],
        (((1,), (1,)), ((), ())),
        preferred_element_type=jnp.float32,
    )
    obuf[slot] = acc + b_ref[...]

    for k in range(_NSPLIT):
        pltpu.make_async_copy(
            obuf.at[slot, pl.ds(k * _RSPL, _RSPL)],
            out_hbm.at[pl.ds(k * _RSPL, _RSPL), pl.ds(j * _VBLK, _VBLK)],
            sem.at[slot],
        ).start(priority=k % 2)

    @pl.when(j == _NV - 1)
    def _drain():
        for i in range(_NBUF):
            pltpu.make_async_copy(
                obuf.at[i],
                out_hbm.at[:, pl.ds(i * _VBLK, _VBLK)],
                sem.at[i],
            ).wait()


def _head(h, head_weight, bias2d):
    return pl.pallas_call(
        _head_body,
        grid=(_NV,),
        in_specs=[
            pl.BlockSpec((B, H), lambda j: (0, 0)),
            pl.BlockSpec((_VBLK, H), lambda j: (j, 0)),
            pl.BlockSpec((1, _VBLK), lambda j: (0, j)),
        ],
        out_specs=pl.BlockSpec(memory_space=pl.ANY),
        out_shape=jax.ShapeDtypeStruct((B, V), jnp.float32),
        scratch_shapes=[
            pltpu.VMEM((_NBUF, B, _VBLK), jnp.float32),
            pltpu.SemaphoreType.DMA((_NBUF,)),
        ],
        compiler_params=pltpu.CompilerParams(
            dimension_semantics=("arbitrary",),
        ),
    )(h, head_weight, bias2d)


def _tail_body(out_in, h_ref, w_ref, b_ref, out_ref):
    del out_in
    acc = lax.dot_general(
        h_ref[...],
        w_ref[...],
        (((1,), (1,)), ((), ())),
        preferred_element_type=jnp.float32,
    )
    out_ref[...] = acc + b_ref[...]


def _tail(logits, h, head_weight, bias2d):
    return pl.pallas_call(
        _tail_body,
        grid=(1,),
        in_specs=[
            pl.BlockSpec(memory_space=pl.ANY),
            pl.BlockSpec((B, H), lambda j: (0, 0)),
            pl.BlockSpec((_VBLK, H), lambda j: (_NV, 0)),
            pl.BlockSpec((1, _VBLK), lambda j: (0, _NV)),
        ],
        out_specs=pl.BlockSpec((B, _VBLK), lambda j: (0, _NV)),
        out_shape=jax.ShapeDtypeStruct((B, V), jnp.float32),
        input_output_aliases={0: 0},
    )(logits, h, head_weight, bias2d)


def kernel(input_ids, embed_weight, head_weight, head_bias):
    last_ids = input_ids[:, -1]
    h = _sc_gather()(embed_weight, last_ids)
    bias2d = head_bias.reshape(1, V)
    logits = _head(h, head_weight, bias2d)
    logits = _tail(logits, h, head_weight, bias2d)
    return logits


# speedup vs baseline: 1.0024x; 1.0019x over previous
import functools

import jax
import jax.numpy as jnp
from jax import lax
from jax.experimental import pallas as pl
from jax.experimental.pallas import tpu as pltpu
from jax.experimental.pallas import tpu_sc as plsc

V = 100000
H = 64
B = 4096


_NC, _NS = 2, 16
_NW = _NC * _NS
_B_PER_W = B // _NW


def _gather_body(table_hbm, idx_hbm, out_hbm, idx_v, rows_v, sem):
    wid = lax.axis_index("s") * _NC + lax.axis_index("c")
    base = wid * _B_PER_W
    pltpu.sync_copy(idx_hbm.at[pl.ds(base, _B_PER_W)], idx_v)
    pltpu.async_copy(table_hbm.at[idx_v], rows_v, sem).wait()
    pltpu.sync_copy(rows_v, out_hbm.at[pl.ds(base, _B_PER_W)])


@functools.cache
def _sc_gather():
    return pl.kernel(
        _gather_body,
        out_type=jax.ShapeDtypeStruct((B, H), jnp.float32),
        mesh=plsc.VectorSubcoreMesh(
            core_axis_name="c", subcore_axis_name="s",
            num_cores=_NC, num_subcores=_NS,
        ),
        scratch_types=[
            pltpu.VMEM((_B_PER_W,), jnp.int32),
            pltpu.VMEM((_B_PER_W, H), jnp.float32),
            pltpu.SemaphoreType.DMA,
        ],
        compiler_params=pltpu.CompilerParams(use_tc_tiling_on_sc=False),
    )


_VBLK = 512
_NBUF = 3
_NSPLIT = 4
_RSPL = B // _NSPLIT
_NV = 195
_VMAIN = _NV * _VBLK
_TAIL = V - _VMAIN


def _head_body(h_ref, w_ref, b_ref, wt_ref, bt_ref, out_hbm, obuf, tbuf, sem, sem_t):
    j = pl.program_id(0)
    slot = lax.rem(j, _NBUF)

    @pl.when(j >= _NBUF)
    def _wait_slot():
        pltpu.make_async_copy(
            obuf.at[slot],
            out_hbm.at[:, pl.ds((j - _NBUF) * _VBLK, _VBLK)],
            sem.at[slot],
        ).wait()

    acc = lax.dot_general(
        h_ref[...],
        w_ref[...],
        (((1,), (1,)), ((), ())),
        preferred_element_type=jnp.float32,
    )
    obuf[slot] = acc + b_ref[...]

    pltpu.make_async_copy(
        obuf.at[slot],
        out_hbm.at[:, pl.ds(j * _VBLK, _VBLK)],
        sem.at[slot],
    ).start()

    @pl.when(j == 0)
    def _tail_compute():
        acc_t = lax.dot_general(
            h_ref[...],
            wt_ref[...],
            (((1,), (1,)), ((), ())),
            preferred_element_type=jnp.float32,
        )
        tbuf[...] = acc_t + bt_ref[...]
        pltpu.make_async_copy(
            tbuf,
            out_hbm.at[:, pl.ds(_VMAIN, _TAIL)],
            sem_t,
        ).start()

    @pl.when(j == _NV - 1)
    def _drain():
        for i in range(_NBUF):
            pltpu.make_async_copy(
                obuf.at[i],
                out_hbm.at[:, pl.ds(i * _VBLK, _VBLK)],
                sem.at[i],
            ).wait()
        pltpu.make_async_copy(
            tbuf,
            out_hbm.at[:, pl.ds(_VMAIN, _TAIL)],
            sem_t,
        ).wait()


def _head(h, head_weight, bias2d, w_tail, b_tail):
    return pl.pallas_call(
        _head_body,
        grid=(_NV,),
        in_specs=[
            pl.BlockSpec((B, H), lambda j: (0, 0)),
            pl.BlockSpec((_VBLK, H), lambda j: (j, 0)),
            pl.BlockSpec((1, _VBLK), lambda j: (0, j)),
            pl.BlockSpec((_TAIL, H), lambda j: (0, 0)),
            pl.BlockSpec((1, _TAIL), lambda j: (0, 0)),
        ],
        out_specs=pl.BlockSpec(memory_space=pl.ANY),
        out_shape=jax.ShapeDtypeStruct((B, V), jnp.float32),
        scratch_shapes=[
            pltpu.VMEM((_NBUF, B, _VBLK), jnp.float32),
            pltpu.VMEM((B, _TAIL), jnp.float32),
            pltpu.SemaphoreType.DMA((_NBUF,)),
            pltpu.SemaphoreType.DMA,
        ],
        compiler_params=pltpu.CompilerParams(
            dimension_semantics=("arbitrary",),
        ),
    )(h, head_weight, bias2d, w_tail, b_tail)


def kernel(input_ids, embed_weight, head_weight, head_bias):
    last_ids = input_ids[:, -1]
    h = _sc_gather()(embed_weight, last_ids)
    bias2d = head_bias.reshape(1, V)
    w_tail = head_weight[_VMAIN:]
    b_tail = bias2d[:, _VMAIN:]
    return _head(h, head_weight, bias2d, w_tail, b_tail)


# speedup vs baseline: 3.1689x; 3.1612x over previous
import functools

import jax
import jax.numpy as jnp
from jax import lax
from jax.experimental import pallas as pl
from jax.experimental.pallas import tpu as pltpu
from jax.experimental.pallas import tpu_sc as plsc

V = 100000
H = 64
B = 4096


_NC, _NS = 2, 16
_NW = _NC * _NS
_B_PER_W = B // _NW


def _gather_body(table_hbm, idx_hbm, out_hbm, idx_v, rows_v, sem):
    wid = lax.axis_index("s") * _NC + lax.axis_index("c")
    base = wid * _B_PER_W
    pltpu.sync_copy(idx_hbm.at[pl.ds(base, _B_PER_W)], idx_v)
    pltpu.async_copy(table_hbm.at[idx_v], rows_v, sem).wait()
    pltpu.sync_copy(rows_v, out_hbm.at[pl.ds(base, _B_PER_W)])


@functools.cache
def _sc_gather():
    return pl.kernel(
        _gather_body,
        out_type=jax.ShapeDtypeStruct((B, H), jnp.float32),
        mesh=plsc.VectorSubcoreMesh(
            core_axis_name="c", subcore_axis_name="s",
            num_cores=_NC, num_subcores=_NS,
        ),
        scratch_types=[
            pltpu.VMEM((_B_PER_W,), jnp.int32),
            pltpu.VMEM((_B_PER_W, H), jnp.float32),
            pltpu.SemaphoreType.DMA,
        ],
        compiler_params=pltpu.CompilerParams(use_tc_tiling_on_sc=False),
    )


_VBLK = 512
_NBUF = 3
_NV = (V + _VBLK - 1) // _VBLK
_TAIL = V - (_NV - 1) * _VBLK


def _head_body(wt_ref, h_ref, b_ref, out_hbm, obuf, sem):
    j = pl.program_id(0)
    slot = lax.rem(j, _NBUF)

    @pl.when(j >= _NBUF)
    def _wait_slot():
        pltpu.make_async_copy(
            obuf.at[slot],
            out_hbm.at[pl.ds((j - _NBUF) * _VBLK, _VBLK)],
            sem.at[slot],
        ).wait()

    acc = lax.dot_general(
        wt_ref[...],
        h_ref[...],
        (((0,), (1,)), ((), ())),
        preferred_element_type=jnp.float32,
    )
    obuf[slot] = acc + b_ref[...]

    @pl.when(j < _NV - 1)
    def _start_full():
        pltpu.make_async_copy(
            obuf.at[slot],
            out_hbm.at[pl.ds(j * _VBLK, _VBLK)],
            sem.at[slot],
        ).start()

    @pl.when(j == _NV - 1)
    def _start_tail_and_drain():
        pltpu.make_async_copy(
            obuf.at[0, pl.ds(0, _TAIL)],
            out_hbm.at[pl.ds((_NV - 1) * _VBLK, _TAIL)],
            sem.at[0],
        ).start()
        for i in (1, 2):
            pltpu.make_async_copy(
                obuf.at[i],
                out_hbm.at[pl.ds(i * _VBLK, _VBLK)],
                sem.at[i],
            ).wait()
        pltpu.make_async_copy(
            obuf.at[0, pl.ds(0, _TAIL)],
            out_hbm.at[pl.ds((_NV - 1) * _VBLK, _TAIL)],
            sem.at[0],
        ).wait()


def _head(wt, h, bias_col):
    return pl.pallas_call(
        _head_body,
        grid=(_NV,),
        in_specs=[
            pl.BlockSpec((H, _VBLK), lambda j: (0, j)),
            pl.BlockSpec((B, H), lambda j: (0, 0)),
            pl.BlockSpec((_VBLK, 1), lambda j: (j, 0)),
        ],
        out_specs=pl.BlockSpec(memory_space=pl.ANY),
        out_shape=jax.ShapeDtypeStruct((V, B), jnp.float32),
        scratch_shapes=[
            pltpu.VMEM((_NBUF, _VBLK, B), jnp.float32),
            pltpu.SemaphoreType.DMA((_NBUF,)),
        ],
        compiler_params=pltpu.CompilerParams(
            dimension_semantics=("arbitrary",),
        ),
    )(wt, h, bias_col)


def kernel(input_ids, embed_weight, head_weight, head_bias):
    last_ids = input_ids[:, -1]
    h = _sc_gather()(embed_weight, last_ids)
    out_t = _head(head_weight.T, h, head_bias.reshape(V, 1))
    return out_t.T
